# baseline (speedup 1.0000x reference)
import functools

import numpy as np
import jax
import jax.numpy as jnp
from jax import lax
from jax.experimental import pallas as pl
from jax.experimental.pallas import tpu as pltpu, tpu_sc as plsc

_PROB = 0.25

_DROPPED_B16 = (7, 6, 3, 2)


@functools.lru_cache(maxsize=None)
def _dropped_ids(b):
    num = 1 if b == 1 else int(b * _PROB)
    if b == 16:
        return _DROPPED_B16[:num]
    with jax.ensure_compile_time_eval(), jax.default_device(jax.devices("cpu")[0]):
        perm = np.asarray(jax.random.permutation(jax.random.key(1), b))
    return tuple(int(x) for x in perm[:num])


def _select(i, table):
    r = table[0]
    for k in range(1, len(table)):
        r = jnp.where(i == k, table[k], r)
    return r


def _sc_zero(emb0, dropped):
    b, t, c, d = emb0.shape
    info = plsc.get_sparse_core_info()
    nw = info.num_cores * info.num_subcores
    wpb = nw // len(dropped)
    tn = t // wpb
    ch = 32
    nch = tn // ch
    mesh = plsc.VectorSubcoreMesh(core_axis_name="c", subcore_axis_name="s")

    zeros = jnp.zeros((ch, c, d), emb0.dtype)

    @functools.partial(
        pl.kernel,
        out_type=jax.ShapeDtypeStruct((b, t, c, d), emb0.dtype),
        mesh=mesh,
        scratch_types=[
            pltpu.VMEM((ch, c, d), emb0.dtype),
            pltpu.VMEM((ch, c, d), emb0.dtype),
            pltpu.SemaphoreType.DMA,
            pltpu.SemaphoreType.DMA,
            pltpu.SemaphoreType.DMA,
            pltpu.SemaphoreType.DMA,
        ],
    )
    def run(in_hbm, z_hbm, out_hbm, b0, b1, i0, i1, o0, o1):
        wid = lax.axis_index("s") * info.num_cores + lax.axis_index("c")
        bw = _select(wid // wpb, dropped)
        h = wid % wpb
        t0 = h * tn
        is_last = h == wpb - 1

        def dst(i):
            return out_hbm.at[bw, pl.ds(t0 + i * ch, ch)]

        pltpu.async_copy(z_hbm.at[pl.ds(0, ch)], b0, i0).wait()
        out_d = [pltpu.async_copy(b0, dst(i), o0) for i in range(nch - 1)]
        for d_ in out_d:
            d_.wait()

        @pl.when(jnp.logical_not(is_last))
        def _full_tail():
            pltpu.async_copy(b0, dst(nch - 1), o0).wait()

        @pl.when(is_last)
        def _partial_tail():
            pltpu.async_copy(
                b0.at[pl.ds(0, ch - 1)],
                out_hbm.at[bw, pl.ds(t0 + (nch - 1) * ch, ch - 1)],
                o0,
            ).wait()
            pltpu.async_copy(
                in_hbm.at[bw, pl.ds(t - 1, 1)], b1.at[pl.ds(0, 1)], i1
            ).wait()
            pltpu.async_copy(
                b1.at[pl.ds(0, 1)], out_hbm.at[bw, pl.ds(t - 1, 1)], o1
            ).wait()

    return run(emb0, zeros)


def _tc_fill(partial_out, emb0, kept):
    b, t, c, d = emb0.shape

    def body(acc_ref, in_ref, out_ref):
        del acc_ref
        out_ref[...] = in_ref[...]

    return pl.pallas_call(
        body,
        grid=(len(kept),),
        in_specs=[
            pl.BlockSpec(memory_space=pltpu.MemorySpace.HBM),
            pl.BlockSpec((1, t, c, d), lambda i: (_select(i, kept), 0, 0, 0)),
        ],
        out_specs=pl.BlockSpec((1, t, c, d), lambda i: (_select(i, kept), 0, 0, 0)),
        out_shape=jax.ShapeDtypeStruct((b, t, c, d), emb0.dtype),
        input_output_aliases={0: 0},
    )(partial_out, emb0)


def kernel(emb0):
    b, t, c, d = emb0.shape
    dropped = tuple(sorted(_dropped_ids(b)))
    kept = tuple(i for i in range(b) if i not in dropped)
    sc_out = _sc_zero(emb0, dropped)
    return _tc_fill(sc_out, emb0, kept)
